# Initial kernel scaffold; baseline (speedup 1.0000x reference)
import jax
import jax.numpy as jnp
from jax.experimental import pallas as pl


def kernel(x):
    raise NotImplementedError("write your pallas kernel here")


# speedup vs baseline: 15.5998x; 15.5998x over previous
import math
from functools import partial

import jax
import jax.numpy as jnp
from jax.experimental import pallas as pl
from jax.experimental.pallas import tpu as pltpu

_LANE = 128
_EXP_MASK = 0x7FFFFFFF
_CHAINS = 2


def _round_up(n, m):
    return ((n + m - 1) // m) * m


def _key_to_f32(t):
    bits = jnp.where(t < jnp.int32(0), t ^ jnp.int32(_EXP_MASK), t)
    return jax.lax.bitcast_convert_type(bits, jnp.float32)


def _count_ge16(data, t):
    d = data.shape[-1]
    one = jnp.int16(1)
    zero = jnp.int16(0)
    acc = jnp.where(data[:, :128] >= t, one, zero)
    for c in range(1, d // 128):
        acc = acc + jnp.where(data[:, 128 * c:128 * (c + 1)] >= t, one, zero)
    return jnp.sum(acc.astype(jnp.float32), axis=-1, keepdims=True)


def _kwta_kernel(x_ref, o_ref, *, k, a):
    x = x_ref[...]
    bits = jax.lax.bitcast_convert_type(x, jnp.int32)
    kf = jnp.float32(k)

    bh = jax.lax.shift_right_arithmetic(bits, 16).astype(jnp.int16)
    negp = bh < jnp.int16(0)
    hi = bh ^ jnp.where(negp, jnp.int16(0x7FFF), jnp.int16(0))
    lo_s = bits.astype(jnp.int16) ^ jnp.where(negp, jnp.int16(0x7FFF),
                                              jnp.int16(-0x8000))

    TB = x.shape[0]
    R = TB // _CHAINS
    his = [hi[c * R:(c + 1) * R] for c in range(_CHAINS)]

    k16 = jnp.int16(k)
    c_pos = _count_ge16(hi, jnp.int16(0)).astype(jnp.int16)
    candh0 = jnp.where(c_pos >= k16, jnp.int16(0), jnp.int16(-0x8000))
    ch0s = tuple(candh0[c * R:(c + 1) * R] for c in range(_CHAINS))

    def step_a(i, cands):
        bit = jax.lax.shift_left(jnp.int32(1), jnp.int32(14) - i
                                 ).astype(jnp.int16)
        out = []
        for c in range(_CHAINS):
            trial = cands[c] | bit
            cnt = _count_ge16(his[c], trial).astype(jnp.int16)
            out.append(jnp.where(cnt >= k16, trial, cands[c]))
        return tuple(out)

    cands_h = jax.lax.fori_loop(0, 15, step_a, ch0s)
    ch16 = jnp.concatenate(cands_h, axis=0)

    candh = ch16.astype(jnp.int32)
    kp = (kf - _count_ge16(hi, (candh + 1).astype(jnp.int16))
          ).astype(jnp.int16)
    act_lo = jnp.where(hi == ch16, lo_s, jnp.int16(-0x8000))
    acts = [act_lo[c * R:(c + 1) * R] for c in range(_CHAINS)]
    kps = tuple(kp[c * R:(c + 1) * R] for c in range(_CHAINS))

    t0 = jnp.int16(0)
    cl0s = tuple(
        jnp.where(_count_ge16(acts[c], t0).astype(jnp.int16) >= kps[c],
                  t0, jnp.int16(-0x8000))
        for c in range(_CHAINS))

    def step_b(i, cands):
        bit = jax.lax.shift_left(jnp.int32(1), jnp.int32(14) - i
                                 ).astype(jnp.int16)
        out = []
        for c in range(_CHAINS):
            trial = cands[c] | bit
            cnt = _count_ge16(acts[c], trial).astype(jnp.int16)
            out.append(jnp.where(cnt >= kps[c], trial, cands[c]))
        return tuple(out)

    cands_l = jax.lax.fori_loop(0, 15, step_b, cl0s)
    cl16 = jnp.concatenate(cands_l, axis=0)
    candl = (cl16.astype(jnp.int32) & 0xFFFF) ^ 0x8000

    cand = jax.lax.shift_left(candh, 16) | candl

    vk = _key_to_f32(cand)
    m = x >= vk
    c_ge = jnp.sum(m.astype(jnp.float32), axis=-1, keepdims=True)
    below = jnp.max(jnp.where(m, jnp.float32(-jnp.inf), x), axis=-1,
                    keepdims=True)
    vk1 = jnp.where(c_ge >= kf + 1.0, vk, below)
    thr = 0.5 * (vk + vk1)

    b = jnp.float32(0.5) - jnp.float32(a) * thr
    o_ref[...] = jnp.clip(jnp.float32(a) * x + b, 0.0, 1.0)


def kernel(x):
    sparsity, hardness = 0.25, 2.0
    orig_shape = x.shape
    B = orig_shape[0]
    D = math.prod(orig_shape[1:])
    k_active = min(math.ceil(sparsity * D), D - 1)

    TB = 1024
    Dp = _round_up(D, _LANE)
    Bp = _round_up(B, TB)

    xf = x.reshape(B, D).astype(jnp.float32)
    if Dp != D:
        xf = jnp.pad(xf, ((0, 0), (0, Dp - D)), constant_values=-jnp.inf)
    if Bp != B:
        xf = jnp.pad(xf, ((0, Bp - B), (0, 0)))

    kfn = partial(_kwta_kernel, k=k_active, a=float(hardness / 6.0))
    out = pl.pallas_call(
        kfn,
        out_shape=jax.ShapeDtypeStruct((Bp, Dp), jnp.float32),
        grid=(Bp // TB,),
        in_specs=[pl.BlockSpec((TB, Dp), lambda i: (i, 0))],
        out_specs=pl.BlockSpec((TB, Dp), lambda i: (i, 0)),
        compiler_params=pltpu.CompilerParams(
            dimension_semantics=("parallel",)),
    )(xf)

    out = out[:B, :D].reshape(orig_shape)
    return out.astype(x.dtype)
